# baseline (speedup 1.0000x reference)
import functools

import jax
import jax.numpy as jnp
from jax import lax
from jax.experimental import pallas as pl
from jax.experimental.pallas import tpu as pltpu
from jax.experimental.pallas import tpu_sc as plsc

N_NODES = 10000
N_EDGES = 160000
N_CH = 128
N_MAX = 8
CUTOFF = 5.0
NORM_FACTOR = 16.0

NC = 2
NS = 16
CH = 16
NCHUNK = N_CH // CH
CPC = NCHUNK // NC

_MESH = dict(core_axis_name="c", subcore_axis_name="s")
_SC_PARAMS = pltpu.CompilerParams(use_tc_tiling_on_sc=False)


_E_T1 = N_EDGES // (NC * NS)
_B1 = 40


_NB1 = _E_T1 // _B1


@functools.partial(
    pl.kernel,
    out_type=jax.ShapeDtypeStruct((N_EDGES, 16), jnp.float32),
    mesh=plsc.VectorSubcoreMesh(**_MESH),
    scratch_types=[
        pltpu.VMEM((_E_T1,), jnp.int32),
        pltpu.VMEM((_E_T1,), jnp.int32),
        pltpu.VMEM((_B1, 16), jnp.float32),
        pltpu.VMEM((_B1, 16), jnp.float32),
        pltpu.VMEM((_B1, 16), jnp.float32),
        pltpu.VMEM((_B1, 16), jnp.float32),
        pltpu.VMEM((_B1, 16), jnp.float32),
        pltpu.VMEM((_B1, 16), jnp.float32),
        pltpu.SemaphoreType.DMA,
        pltpu.SemaphoreType.DMA,
        pltpu.SemaphoreType.DMA,
        pltpu.SemaphoreType.DMA,
    ],
    compiler_params=_SC_PARAMS,
)
def _gather_coords(coordp, idxi_h, idxj_h, rij_h,
                   idxi_t, idxj_t, ci0, ci1, cj0, cj1, rb0, rb1,
                   g0, g1, w0, w1):
    cib = [ci0, ci1]
    cjb = [cj0, cj1]
    rijb = [rb0, rb1]
    gsem = [g0, g1]
    wsem = [w0, w1]
    cid = lax.axis_index("c")
    sid = lax.axis_index("s")
    base0 = (sid * NC + cid) * _E_T1
    pltpu.sync_copy(idxi_h.at[pl.ds(base0, _E_T1)], idxi_t)
    pltpu.sync_copy(idxj_h.at[pl.ds(base0, _E_T1)], idxj_t)

    def gdescs(p, n):
        lo = n * _B1
        return [
            pltpu.make_async_copy(coordp.at[idxi_t.at[pl.ds(lo, _B1)]],
                                  cib[p], gsem[p]),
            pltpu.make_async_copy(coordp.at[idxj_t.at[pl.ds(lo, _B1)]],
                                  cjb[p], gsem[p]),
        ]

    def wdesc(p, n):
        return pltpu.make_async_copy(rijb[p], rij_h.at[pl.ds(base0 + n * _B1, _B1)],
                                     wsem[p])

    def step(b, p):
        q = 1 - p
        for dsc in gdescs(p, b):
            dsc.wait()

        @pl.when(b <= _NB1 - 2)
        def _():
            for dsc in gdescs(q, b + 1):
                dsc.start()

        @pl.when(b >= 2)
        def _():
            wdesc(p, b - 2).wait()

        @plsc.parallel_loop(0, _B1, 1, unroll=4)
        def _sub(e):
            rijb[p][e, :] = cjb[p][e, :] - cib[p][e, :]
        wdesc(p, b).start()

    for dsc in gdescs(0, 0):
        dsc.start()

    def pair(g, carry):
        step(2 * g, 0)
        step(2 * g + 1, 1)
        return carry

    lax.fori_loop(0, _NB1 // 2, pair, 0)
    step(_NB1 - 1, (_NB1 - 1) % 2)
    wdesc((_NB1 - 2) % 2, _NB1 - 2).wait()
    wdesc((_NB1 - 1) % 2, _NB1 - 1).wait()


_BE = 2000


def _tc_filter_body(rij_ref, w01_ref, fn_ref, u_ref):
    rT = rij_ref[...].T
    rx = rT[0:1, :]
    ry = rT[1:2, :]
    rz = rT[2:3, :]
    d = jnp.sqrt(rx * rx + ry * ry + rz * rz + 1e-12)
    inv = 1.0 / d
    nvec = (jnp.arange(1, N_MAX + 1, dtype=jnp.int32)).astype(jnp.float32)
    arg = nvec[:, None] * ((jnp.pi / CUTOFF) * d)
    cut = 0.5 * (jnp.cos(jnp.pi / CUTOFF * d) + 1.0) * (d < CUTOFF).astype(jnp.float32)
    scale = jnp.sqrt(2.0 / CUTOFF) * inv * cut * (1.0 / NORM_FACTOR)
    rbf = jnp.sin(arg) * scale
    fn_ref[...] = lax.dot_general(
        rbf, w01_ref[...], (((0,), (0,)), ((), ())),
        preferred_element_type=jnp.float32)
    u3 = jnp.concatenate([rx * inv, ry * inv, rz * inv], axis=0)
    sel = (lax.broadcasted_iota(jnp.int32, (3, 16), 1)
           == lax.broadcasted_iota(jnp.int32, (3, 16), 0)).astype(jnp.float32)
    u_ref[...] = lax.dot_general(
        u3, sel, (((0,), (0,)), ((), ())), preferred_element_type=jnp.float32)


def _tc_filters(rij, w01):
    grid = (N_EDGES // _BE,)
    return pl.pallas_call(
        _tc_filter_body,
        grid=grid,
        in_specs=[
            pl.BlockSpec((_BE, 16), lambda i: (i, 0)),
            pl.BlockSpec((N_MAX, 2 * N_CH), lambda i: (0, 0)),
        ],
        out_specs=[
            pl.BlockSpec((_BE, 2 * N_CH), lambda i: (i, 0)),
            pl.BlockSpec((_BE, 16), lambda i: (i, 0)),
        ],
        out_shape=[
            jax.ShapeDtypeStruct((N_EDGES, 2 * N_CH), jnp.float32),
            jax.ShapeDtypeStruct((N_EDGES, 16), jnp.float32),
        ],
    )(rij, w01)


_BN = 400


def _tt_body(t0_ref, t1_ref, tt_ref):
    r = lax.broadcasted_iota(jnp.int32, (3 * CH, 3 * CH), 0)
    q = lax.broadcasted_iota(jnp.int32, (3 * CH, 3 * CH), 1)
    perm = ((r // 3 == q % CH) & (r % 3 == q // CH)).astype(jnp.float32)
    for ck in range(NCHUNK):
        tt_ref[ck, :, :CH] = t0_ref[:, ck * CH:(ck + 1) * CH]
        tt_ref[ck, :, CH:] = lax.dot_general(
            t1_ref[:, ck, :], perm, (((1,), (0,)), ((), ())),
            preferred_element_type=jnp.float32)


def _tt_build(t0, t1v):
    return pl.pallas_call(
        _tt_body,
        grid=(N_NODES // _BN,),
        in_specs=[
            pl.BlockSpec((_BN, N_CH), lambda i: (i, 0)),
            pl.BlockSpec((_BN, NCHUNK, 3 * CH), lambda i: (i, 0, 0)),
        ],
        out_specs=pl.BlockSpec((NCHUNK, _BN, 4 * CH), lambda i: (0, i, 0)),
        out_shape=jax.ShapeDtypeStruct((NCHUNK, N_NODES, 4 * CH), jnp.float32),
    )(t0, t1v)


def _out_body(acc_ref, out0_ref, out1_ref):
    r = lax.broadcasted_iota(jnp.int32, (3 * CH, 3 * CH), 0)
    q = lax.broadcasted_iota(jnp.int32, (3 * CH, 3 * CH), 1)
    iperm = ((r // CH == q % 3) & (r % CH == q // 3)).astype(jnp.float32)
    for ck in range(NCHUNK):
        out0_ref[:, ck * CH:(ck + 1) * CH] = acc_ref[ck, :, :CH]
        out1_ref[:, ck * 3 * CH:(ck + 1) * 3 * CH] = lax.dot_general(
            acc_ref[ck, :, CH:], iperm, (((1,), (0,)), ((), ())),
            preferred_element_type=jnp.float32)


def _out_build(acc):
    return pl.pallas_call(
        _out_body,
        grid=(N_NODES // _BN,),
        in_specs=[pl.BlockSpec((NCHUNK, _BN, 4 * CH), lambda i: (0, i, 0))],
        out_specs=[
            pl.BlockSpec((_BN, N_CH), lambda i: (i, 0)),
            pl.BlockSpec((_BN, 3 * N_CH), lambda i: (i, 0)),
        ],
        out_shape=[
            jax.ShapeDtypeStruct((N_NODES, N_CH), jnp.float32),
            jax.ShapeDtypeStruct((N_NODES, 3 * N_CH), jnp.float32),
        ],
    )(acc)


_EPT = N_EDGES // NS
_B3 = 80
_NBLK = _EPT // _B3
_STRIPE = N_NODES // NS
_ZROWS = 125


@functools.partial(
    pl.kernel,
    out_type=jax.ShapeDtypeStruct((NCHUNK, N_NODES, 4 * CH), jnp.float32),
    mesh=plsc.VectorSubcoreMesh(**_MESH),
    scratch_types=[
        pltpu.VMEM_SHARED((N_NODES, 4 * CH), jnp.float32),
        pltpu.VMEM((_NBLK, _B3), jnp.int32),
        pltpu.VMEM((_EPT,), jnp.int32),
        pltpu.VMEM((_B3,), jnp.int32),
        pltpu.VMEM((_B3,), jnp.int32),
        pltpu.VMEM((_B3, 4 * CH), jnp.float32),
        pltpu.VMEM((_B3, 4 * CH), jnp.float32),
        pltpu.VMEM((_B3, 2 * CH), jnp.float32),
        pltpu.VMEM((_B3, 2 * CH), jnp.float32),
        pltpu.VMEM((_B3, 16), jnp.float32),
        pltpu.VMEM((_B3, 16), jnp.float32),
        pltpu.VMEM((_B3, 4 * CH), jnp.float32),
        pltpu.VMEM((_B3, 4 * CH), jnp.float32),
        pltpu.VMEM((_B3, 4 * CH), jnp.float32),
        pltpu.VMEM((_B3, 4 * CH), jnp.float32),
        pltpu.VMEM((_ZROWS, 4 * CH), jnp.float32),
        pltpu.SemaphoreType.DMA,
        pltpu.SemaphoreType.DMA,
        pltpu.SemaphoreType.DMA,
        pltpu.SemaphoreType.DMA,
        pltpu.SemaphoreType.DMA,
        pltpu.SemaphoreType.DMA,
    ],
    compiler_params=_SC_PARAMS,
)
def _sc_aggregate(tt_h, fn_h, u_h, idxi2_h, idxj_h, acc_h,
                  acc_sh, idxi_v2, idxj_full, ij0, ij1, tb0_, tb1_,
                  fb0_, fb1_, ub0, ub1, m0, m1, m2, m3,
                  zbuf, gs0, gs1, ss0, ss1, ss2, ss3):
    idxj2 = [ij0, ij1]
    tb = [tb0_, tb1_]
    f01b = [fb0_, fb1_]
    ub = [ub0, ub1]
    msgb = [m0, m1, m2, m3]
    gsem = [gs0, gs1]
    ssem = [ss0, ss1, ss2, ss3]
    cid = lax.axis_index("c")
    sid = lax.axis_index("s")

    pltpu.sync_copy(idxi2_h.at[pl.ds(sid * _NBLK, _NBLK)], idxi_v2)
    pltpu.sync_copy(idxj_h.at[pl.ds(sid * _EPT, _EPT)], idxj_full)

    zero16 = jnp.zeros((16,), jnp.float32)

    def zrow(i, carry):
        for g in range(4 * CH // 16):
            zbuf[i, pl.ds(g * 16, 16)] = zero16
        return carry

    lax.fori_loop(0, _ZROWS, zrow, 0)

    def zero_stripe():
        for q in range(_STRIPE // _ZROWS):
            pltpu.sync_copy(zbuf, acc_sh.at[pl.ds(sid * _STRIPE + q * _ZROWS, _ZROWS)])

    zero_stripe()
    plsc.subcore_barrier()

    def chunk_body(k, carry):
        ck = cid * CPC + k
        coff = ck * CH
        joff = ck * N_NODES

        def gdescs(q, n):
            base = sid * _EPT + n * _B3
            return [
                pltpu.make_async_copy(tt_h.at[idxj2[q]], tb[q], gsem[q]),
                pltpu.make_async_copy(
                    fn_h.at[pl.ds(base, _B3), pl.ds(2 * coff, 2 * CH)],
                    f01b[q], gsem[q]),
                pltpu.make_async_copy(u_h.at[pl.ds(base, _B3)], ub[q], gsem[q]),
            ]

        def sdesc(r, n):
            return pltpu.make_async_copy(msgb[r], acc_sh.at[idxi_v2.at[n]],
                                         ssem[r])

        def prep_issue(n, q):
            lo = n * _B3
            for i in range(_B3 // 16):
                idxj2[q][pl.ds(i * 16, 16)] = (
                    idxj_full[pl.ds(lo + i * 16, 16)] + joff)
            for dsc in gdescs(q, n):
                dsc.start()

        def compute(p, r):
            tbp, fbp, uu, mm = tb[p], f01b[p], ub[p], msgb[r]
            lane0 = jnp.zeros((16, 1), jnp.int32)
            lane1 = lane0 + 1
            lane2 = lane0 + 2
            gdn = lax.GatherDimensionNumbers(
                offset_dims=(), collapsed_slice_dims=(0,),
                start_index_map=(0,))

            def bcast(uv, lanes):
                return lax.gather(
                    uv, lanes, gdn, (1,),
                    mode=lax.GatherScatterMode.PROMISE_IN_BOUNDS)

            @plsc.parallel_loop(0, _B3, 1, unroll=8)
            def _edge(e):
                uv = uu[e, :]
                u0 = bcast(uv, lane0)
                u1 = bcast(uv, lane1)
                u2 = bcast(uv, lane2)
                for g in range(CH // 16):
                    sl = pl.ds(g * 16, 16)
                    t0v = tbp[e, sl]
                    f0 = fbp[e, sl]
                    f1 = fbp[e, pl.ds(CH + g * 16, 16)]
                    t1x = tbp[e, pl.ds(CH + g * 16, 16)]
                    t1y = tbp[e, pl.ds(2 * CH + g * 16, 16)]
                    t1z = tbp[e, pl.ds(3 * CH + g * 16, 16)]
                    s_ = t1x * u0 + t1y * u1 + t1z * u2
                    a = t0v * f1
                    mm[e, sl] = t0v * f0 + s_ * f1
                    mm[e, pl.ds(CH + g * 16, 16)] = a * u0 + t1x * f0
                    mm[e, pl.ds(2 * CH + g * 16, 16)] = a * u1 + t1y * f0
                    mm[e, pl.ds(3 * CH + g * 16, 16)] = a * u2 + t1z * f0

        def step(b, r):
            p = r & 1
            q = (r + 1) & 1

            @pl.when(b >= 3)
            def _():
                sdesc((r + 1) % 4, b - 3).wait()

            for dsc in gdescs(p, b):
                dsc.wait()

            @pl.when(b <= _NBLK - 2)
            def _():
                prep_issue(b + 1, q)

            compute(p, r)
            sdesc(r, b).start(add=True)

        prep_issue(0, 0)

        def group(g, c2):
            for rr in range(4):
                step(4 * g + rr, rr)
            return c2

        lax.fori_loop(0, (_NBLK - 1) // 4, group, 0)
        step(_NBLK - 1, (_NBLK - 1) % 4)
        sdesc((_NBLK - 3) % 4, _NBLK - 3).wait()
        sdesc((_NBLK - 2) % 4, _NBLK - 2).wait()
        sdesc((_NBLK - 1) % 4, _NBLK - 1).wait()

        plsc.subcore_barrier()
        for q in range(_STRIPE // _ZROWS):
            row = sid * _STRIPE + q * _ZROWS
            pltpu.sync_copy(acc_sh.at[pl.ds(row, _ZROWS)],
                            acc_h.at[ck, pl.ds(row, _ZROWS)])
        plsc.subcore_barrier()
        zero_stripe()
        plsc.subcore_barrier()
        return carry

    lax.fori_loop(0, CPC, chunk_body, 0)


def kernel(input_tensor_0, input_tensor_1, W0, W1, coordinate, edge_index,
           atomic_number):
    del atomic_number
    idx_i = edge_index[0].astype(jnp.int32)
    idx_j = edge_index[1].astype(jnp.int32)
    coordp = jnp.pad(coordinate, ((0, 0), (0, 13)))
    rij = _gather_coords(coordp, idx_i, idx_j)
    w01 = jnp.stack(
        [W0.T.reshape(N_MAX, NCHUNK, CH), W1.T.reshape(N_MAX, NCHUNK, CH)],
        axis=2).reshape(N_MAX, 2 * N_CH)
    fn, uij = _tc_filters(rij, w01)
    t1v = input_tensor_1.reshape(N_NODES, NCHUNK, 3 * CH)
    tt = _tt_build(input_tensor_0, t1v).reshape(NCHUNK * N_NODES, 4 * CH)
    acc = _sc_aggregate(tt, fn, uij,
                        idx_i.reshape(N_EDGES // _B3, _B3), idx_j)
    out0, out1f = _out_build(acc)
    return out0, out1f.reshape(N_NODES, N_CH, 3)


# speedup vs baseline: 1.0009x; 1.0009x over previous
import functools

import jax
import jax.numpy as jnp
from jax import lax
from jax.experimental import pallas as pl
from jax.experimental.pallas import tpu as pltpu
from jax.experimental.pallas import tpu_sc as plsc

N_NODES = 10000
N_EDGES = 160000
N_CH = 128
N_MAX = 8
CUTOFF = 5.0
NORM_FACTOR = 16.0

NC = 2
NS = 16
CH = 16
NCHUNK = N_CH // CH
CPC = NCHUNK // NC

_MESH = dict(core_axis_name="c", subcore_axis_name="s")
_SC_PARAMS = pltpu.CompilerParams(use_tc_tiling_on_sc=False)


_E_T1 = N_EDGES // (NC * NS)
_B1 = 40


_NB1 = _E_T1 // _B1


@functools.partial(
    pl.kernel,
    out_type=jax.ShapeDtypeStruct((N_EDGES, 16), jnp.float32),
    mesh=plsc.VectorSubcoreMesh(**_MESH),
    scratch_types=[
        pltpu.VMEM((_E_T1,), jnp.int32),
        pltpu.VMEM((_E_T1,), jnp.int32),
        pltpu.VMEM((_B1, 16), jnp.float32),
        pltpu.VMEM((_B1, 16), jnp.float32),
        pltpu.VMEM((_B1, 16), jnp.float32),
        pltpu.VMEM((_B1, 16), jnp.float32),
        pltpu.VMEM((_B1, 16), jnp.float32),
        pltpu.VMEM((_B1, 16), jnp.float32),
        pltpu.SemaphoreType.DMA,
        pltpu.SemaphoreType.DMA,
        pltpu.SemaphoreType.DMA,
        pltpu.SemaphoreType.DMA,
    ],
    compiler_params=_SC_PARAMS,
)
def _gather_coords(coordp, idxi_h, idxj_h, rij_h,
                   idxi_t, idxj_t, ci0, ci1, cj0, cj1, rb0, rb1,
                   g0, g1, w0, w1):
    cib = [ci0, ci1]
    cjb = [cj0, cj1]
    rijb = [rb0, rb1]
    gsem = [g0, g1]
    wsem = [w0, w1]
    cid = lax.axis_index("c")
    sid = lax.axis_index("s")
    base0 = (sid * NC + cid) * _E_T1
    pltpu.sync_copy(idxi_h.at[pl.ds(base0, _E_T1)], idxi_t)
    pltpu.sync_copy(idxj_h.at[pl.ds(base0, _E_T1)], idxj_t)

    def gdescs(p, n):
        lo = n * _B1
        return [
            pltpu.make_async_copy(coordp.at[idxi_t.at[pl.ds(lo, _B1)]],
                                  cib[p], gsem[p]),
            pltpu.make_async_copy(coordp.at[idxj_t.at[pl.ds(lo, _B1)]],
                                  cjb[p], gsem[p]),
        ]

    def wdesc(p, n):
        return pltpu.make_async_copy(rijb[p], rij_h.at[pl.ds(base0 + n * _B1, _B1)],
                                     wsem[p])

    def step(b, p):
        q = 1 - p
        for dsc in gdescs(p, b):
            dsc.wait()

        @pl.when(b <= _NB1 - 2)
        def _():
            for dsc in gdescs(q, b + 1):
                dsc.start()

        @pl.when(b >= 2)
        def _():
            wdesc(p, b - 2).wait()

        @plsc.parallel_loop(0, _B1, 1, unroll=4)
        def _sub(e):
            rijb[p][e, :] = cjb[p][e, :] - cib[p][e, :]
        wdesc(p, b).start()

    for dsc in gdescs(0, 0):
        dsc.start()

    def pair(g, carry):
        step(2 * g, 0)
        step(2 * g + 1, 1)
        return carry

    lax.fori_loop(0, _NB1 // 2, pair, 0)
    step(_NB1 - 1, (_NB1 - 1) % 2)
    wdesc((_NB1 - 2) % 2, _NB1 - 2).wait()
    wdesc((_NB1 - 1) % 2, _NB1 - 1).wait()


_BE = 2000


def _tc_filter_body(rij_ref, w01_ref, fn_ref, u_ref):
    rT = rij_ref[...].T
    rx = rT[0:1, :]
    ry = rT[1:2, :]
    rz = rT[2:3, :]
    d = jnp.sqrt(rx * rx + ry * ry + rz * rz + 1e-12)
    inv = 1.0 / d
    nvec = (jnp.arange(1, N_MAX + 1, dtype=jnp.int32)).astype(jnp.float32)
    arg = nvec[:, None] * ((jnp.pi / CUTOFF) * d)
    cut = 0.5 * (jnp.cos(jnp.pi / CUTOFF * d) + 1.0) * (d < CUTOFF).astype(jnp.float32)
    scale = jnp.sqrt(2.0 / CUTOFF) * inv * cut * (1.0 / NORM_FACTOR)
    rbf = jnp.sin(arg) * scale
    fn_ref[...] = lax.dot_general(
        rbf, w01_ref[...], (((0,), (0,)), ((), ())),
        preferred_element_type=jnp.float32)
    u3 = jnp.concatenate([rx * inv, ry * inv, rz * inv], axis=0)
    sel = (lax.broadcasted_iota(jnp.int32, (3, 16), 1)
           == lax.broadcasted_iota(jnp.int32, (3, 16), 0)).astype(jnp.float32)
    u_ref[...] = lax.dot_general(
        u3, sel, (((0,), (0,)), ((), ())), preferred_element_type=jnp.float32)


def _tc_filters(rij, w01):
    grid = (N_EDGES // _BE,)
    return pl.pallas_call(
        _tc_filter_body,
        grid=grid,
        in_specs=[
            pl.BlockSpec((_BE, 16), lambda i: (i, 0)),
            pl.BlockSpec((N_MAX, 2 * N_CH), lambda i: (0, 0)),
        ],
        out_specs=[
            pl.BlockSpec((_BE, 2 * N_CH), lambda i: (i, 0)),
            pl.BlockSpec((_BE, 16), lambda i: (i, 0)),
        ],
        out_shape=[
            jax.ShapeDtypeStruct((N_EDGES, 2 * N_CH), jnp.float32),
            jax.ShapeDtypeStruct((N_EDGES, 16), jnp.float32),
        ],
    )(rij, w01)


_BN = 400


def _tt_body(t0_ref, t1_ref, tt_ref):
    r = lax.broadcasted_iota(jnp.int32, (3 * CH, 3 * CH), 0)
    q = lax.broadcasted_iota(jnp.int32, (3 * CH, 3 * CH), 1)
    perm = ((r // 3 == q % CH) & (r % 3 == q // CH)).astype(jnp.float32)
    for ck in range(NCHUNK):
        tt_ref[ck, :, :CH] = t0_ref[:, ck * CH:(ck + 1) * CH]
        tt_ref[ck, :, CH:] = lax.dot_general(
            t1_ref[:, ck, :], perm, (((1,), (0,)), ((), ())),
            preferred_element_type=jnp.float32)


def _tt_build(t0, t1v):
    return pl.pallas_call(
        _tt_body,
        grid=(N_NODES // _BN,),
        in_specs=[
            pl.BlockSpec((_BN, N_CH), lambda i: (i, 0)),
            pl.BlockSpec((_BN, NCHUNK, 3 * CH), lambda i: (i, 0, 0)),
        ],
        out_specs=pl.BlockSpec((NCHUNK, _BN, 4 * CH), lambda i: (0, i, 0)),
        out_shape=jax.ShapeDtypeStruct((NCHUNK, N_NODES, 4 * CH), jnp.float32),
    )(t0, t1v)


def _out_body(acc_ref, out0_ref, out1_ref):
    r = lax.broadcasted_iota(jnp.int32, (3 * CH, 3 * CH), 0)
    q = lax.broadcasted_iota(jnp.int32, (3 * CH, 3 * CH), 1)
    iperm = ((r // CH == q % 3) & (r % CH == q // 3)).astype(jnp.float32)
    for ck in range(NCHUNK):
        out0_ref[:, ck * CH:(ck + 1) * CH] = acc_ref[ck, :, :CH]
        out1_ref[:, ck * 3 * CH:(ck + 1) * 3 * CH] = lax.dot_general(
            acc_ref[ck, :, CH:], iperm, (((1,), (0,)), ((), ())),
            preferred_element_type=jnp.float32)


def _out_build(acc):
    return pl.pallas_call(
        _out_body,
        grid=(N_NODES // _BN,),
        in_specs=[pl.BlockSpec((NCHUNK, _BN, 4 * CH), lambda i: (0, i, 0))],
        out_specs=[
            pl.BlockSpec((_BN, N_CH), lambda i: (i, 0)),
            pl.BlockSpec((_BN, 3 * N_CH), lambda i: (i, 0)),
        ],
        out_shape=[
            jax.ShapeDtypeStruct((N_NODES, N_CH), jnp.float32),
            jax.ShapeDtypeStruct((N_NODES, 3 * N_CH), jnp.float32),
        ],
    )(acc)


_EPT = N_EDGES // NS
_B3 = 80
_NBLK = _EPT // _B3
_STRIPE = N_NODES // NS
_ZROWS = 125


@functools.partial(
    pl.kernel,
    out_type=jax.ShapeDtypeStruct((NCHUNK, N_NODES, 4 * CH), jnp.float32),
    mesh=plsc.VectorSubcoreMesh(**_MESH),
    scratch_types=[
        pltpu.VMEM_SHARED((N_NODES, 4 * CH), jnp.float32),
        pltpu.VMEM((_NBLK, _B3), jnp.int32),
        pltpu.VMEM((_EPT,), jnp.int32),
        pltpu.VMEM((_B3,), jnp.int32),
        pltpu.VMEM((_B3,), jnp.int32),
        pltpu.VMEM((_B3, 4 * CH), jnp.float32),
        pltpu.VMEM((_B3, 4 * CH), jnp.float32),
        pltpu.VMEM((_B3, 2 * CH), jnp.float32),
        pltpu.VMEM((_B3, 2 * CH), jnp.float32),
        pltpu.VMEM((_B3, 16), jnp.float32),
        pltpu.VMEM((_B3, 16), jnp.float32),
        pltpu.VMEM((_B3, 4 * CH), jnp.float32),
        pltpu.VMEM((_B3, 4 * CH), jnp.float32),
        pltpu.VMEM((_B3, 4 * CH), jnp.float32),
        pltpu.VMEM((_B3, 4 * CH), jnp.float32),
        pltpu.VMEM((_ZROWS, 4 * CH), jnp.float32),
        pltpu.SemaphoreType.DMA,
        pltpu.SemaphoreType.DMA,
        pltpu.SemaphoreType.DMA,
        pltpu.SemaphoreType.DMA,
        pltpu.SemaphoreType.DMA,
        pltpu.SemaphoreType.DMA,
    ],
    compiler_params=_SC_PARAMS,
)
def _sc_aggregate(tt_h, fn_h, u_h, idxi2_h, idxj_h, acc_h,
                  acc_sh, idxi_v2, idxj_full, ij0, ij1, tb0_, tb1_,
                  fb0_, fb1_, ub0, ub1, m0, m1, m2, m3,
                  zbuf, gs0, gs1, ss0, ss1, ss2, ss3):
    idxj2 = [ij0, ij1]
    tb = [tb0_, tb1_]
    f01b = [fb0_, fb1_]
    ub = [ub0, ub1]
    msgb = [m0, m1, m2, m3]
    gsem = [gs0, gs1]
    ssem = [ss0, ss1, ss2, ss3]
    cid = lax.axis_index("c")
    sid = lax.axis_index("s")

    pltpu.sync_copy(idxi2_h.at[pl.ds(sid * _NBLK, _NBLK)], idxi_v2)
    pltpu.sync_copy(idxj_h.at[pl.ds(sid * _EPT, _EPT)], idxj_full)

    zero16 = jnp.zeros((16,), jnp.float32)

    def zrow(i, carry):
        for g in range(4 * CH // 16):
            zbuf[i, pl.ds(g * 16, 16)] = zero16
        return carry

    lax.fori_loop(0, _ZROWS, zrow, 0)

    def zero_stripe():
        for q in range(_STRIPE // _ZROWS):
            pltpu.sync_copy(zbuf, acc_sh.at[pl.ds(sid * _STRIPE + q * _ZROWS, _ZROWS)])

    zero_stripe()
    plsc.subcore_barrier()

    def chunk_body(k, carry):
        ck = cid * CPC + k
        coff = ck * CH
        joff = ck * N_NODES

        def gdescs(q, n):
            base = sid * _EPT + n * _B3
            return [
                pltpu.make_async_copy(tt_h.at[idxj2[q]], tb[q], gsem[q]),
                pltpu.make_async_copy(
                    fn_h.at[pl.ds(base, _B3), pl.ds(2 * coff, 2 * CH)],
                    f01b[q], gsem[q]),
                pltpu.make_async_copy(u_h.at[pl.ds(base, _B3)], ub[q], gsem[q]),
            ]

        def sdesc(r, n):
            return pltpu.make_async_copy(msgb[r], acc_sh.at[idxi_v2.at[n]],
                                         ssem[r])

        def prep_issue(n, q):
            lo = n * _B3
            for i in range(_B3 // 16):
                idxj2[q][pl.ds(i * 16, 16)] = (
                    idxj_full[pl.ds(lo + i * 16, 16)] + joff)
            for dsc in gdescs(q, n):
                dsc.start()

        def compute(p, r):
            tbp, fbp, uu, mm = tb[p], f01b[p], ub[p], msgb[r]
            lane0 = jnp.zeros((16, 1), jnp.int32)
            lane1 = lane0 + 1
            lane2 = lane0 + 2
            gdn = lax.GatherDimensionNumbers(
                offset_dims=(), collapsed_slice_dims=(0,),
                start_index_map=(0,))

            def bcast(uv, lanes):
                return lax.gather(
                    uv, lanes, gdn, (1,),
                    mode=lax.GatherScatterMode.PROMISE_IN_BOUNDS)

            @plsc.parallel_loop(0, _B3, 1, unroll=4)
            def _edge(e):
                uv = uu[e, :]
                u0 = bcast(uv, lane0)
                u1 = bcast(uv, lane1)
                u2 = bcast(uv, lane2)
                for g in range(CH // 16):
                    sl = pl.ds(g * 16, 16)
                    t0v = tbp[e, sl]
                    f0 = fbp[e, sl]
                    f1 = fbp[e, pl.ds(CH + g * 16, 16)]
                    t1x = tbp[e, pl.ds(CH + g * 16, 16)]
                    t1y = tbp[e, pl.ds(2 * CH + g * 16, 16)]
                    t1z = tbp[e, pl.ds(3 * CH + g * 16, 16)]
                    s_ = t1x * u0 + t1y * u1 + t1z * u2
                    a = t0v * f1
                    mm[e, sl] = t0v * f0 + s_ * f1
                    mm[e, pl.ds(CH + g * 16, 16)] = a * u0 + t1x * f0
                    mm[e, pl.ds(2 * CH + g * 16, 16)] = a * u1 + t1y * f0
                    mm[e, pl.ds(3 * CH + g * 16, 16)] = a * u2 + t1z * f0

        def step(b, r):
            p = r & 1
            q = (r + 1) & 1

            @pl.when(b >= 3)
            def _():
                sdesc((r + 1) % 4, b - 3).wait()

            for dsc in gdescs(p, b):
                dsc.wait()

            @pl.when(b <= _NBLK - 2)
            def _():
                prep_issue(b + 1, q)

            compute(p, r)
            sdesc(r, b).start(add=True)

        prep_issue(0, 0)

        def group(g, c2):
            for rr in range(4):
                step(4 * g + rr, rr)
            return c2

        lax.fori_loop(0, (_NBLK - 1) // 4, group, 0)
        step(_NBLK - 1, (_NBLK - 1) % 4)
        sdesc((_NBLK - 3) % 4, _NBLK - 3).wait()
        sdesc((_NBLK - 2) % 4, _NBLK - 2).wait()
        sdesc((_NBLK - 1) % 4, _NBLK - 1).wait()

        plsc.subcore_barrier()
        for q in range(_STRIPE // _ZROWS):
            row = sid * _STRIPE + q * _ZROWS
            pltpu.sync_copy(acc_sh.at[pl.ds(row, _ZROWS)],
                            acc_h.at[ck, pl.ds(row, _ZROWS)])
        plsc.subcore_barrier()
        zero_stripe()
        plsc.subcore_barrier()
        return carry

    lax.fori_loop(0, CPC, chunk_body, 0)


def kernel(input_tensor_0, input_tensor_1, W0, W1, coordinate, edge_index,
           atomic_number):
    del atomic_number
    idx_i = edge_index[0].astype(jnp.int32)
    idx_j = edge_index[1].astype(jnp.int32)
    coordp = jnp.pad(coordinate, ((0, 0), (0, 13)))
    rij = _gather_coords(coordp, idx_i, idx_j)
    w01 = jnp.stack(
        [W0.T.reshape(N_MAX, NCHUNK, CH), W1.T.reshape(N_MAX, NCHUNK, CH)],
        axis=2).reshape(N_MAX, 2 * N_CH)
    fn, uij = _tc_filters(rij, w01)
    t1v = input_tensor_1.reshape(N_NODES, NCHUNK, 3 * CH)
    tt = _tt_build(input_tensor_0, t1v).reshape(NCHUNK * N_NODES, 4 * CH)
    acc = _sc_aggregate(tt, fn, uij,
                        idx_i.reshape(N_EDGES // _B3, _B3), idx_j)
    out0, out1f = _out_build(acc)
    return out0, out1f.reshape(N_NODES, N_CH, 3)
